# baseline (speedup 1.0000x reference)
import math

import numpy as np
import jax
import jax.numpy as jnp
from jax.experimental import pallas as pl
from jax.experimental.pallas import tpu as pltpu

L = 2048
DM = 768
H = 12
DK = 64
U = min(5 * int(np.ceil(np.log(L))), L)
EPS = 1e-6
NEG = float(np.float32(-3.0e38))


def _rotl32(x, d):
    return ((x << np.uint32(d)) | (x >> np.uint32(32 - d))).astype(np.uint32)


def _threefry2x32(k0, k1, x0, x1):
    rot = [(13, 15, 26, 6), (17, 29, 16, 24)]
    ks = [np.uint32(k0), np.uint32(k1),
          np.uint32(np.uint32(k0) ^ np.uint32(k1) ^ np.uint32(0x1BD11BDA))]
    x0 = (x0 + ks[0]).astype(np.uint32)
    x1 = (x1 + ks[1]).astype(np.uint32)
    for i in range(5):
        for r in rot[i % 2]:
            x0 = (x0 + x1).astype(np.uint32)
            x1 = _rotl32(x1, r)
            x1 = (x1 ^ x0).astype(np.uint32)
        x0 = (x0 + ks[(i + 1) % 3]).astype(np.uint32)
        x1 = (x1 + ks[(i + 2) % 3] + np.uint32(i + 1)).astype(np.uint32)
    return x0, x1


def _sample_indices() -> np.ndarray:
    b1, b2 = _threefry2x32(0, 42, np.zeros(2, np.uint32),
                           np.arange(2, dtype=np.uint32))
    n = L * U
    h1, h2 = _threefry2x32(b1[1], b2[1], np.zeros(n, np.uint32),
                           np.arange(n, dtype=np.uint32))
    bits = (h1 ^ h2).astype(np.uint32)
    return (bits % np.uint32(L)).astype(np.int32).reshape(L, U)


def _sample_counts_T() -> np.ndarray:
    idx = _sample_indices()
    cnt = np.zeros((L, L), np.float32)
    np.add.at(cnt, (np.arange(L)[:, None], idx), 1.0)
    return np.ascontiguousarray(cnt.T)


_CNT_T = _sample_counts_T()


def _attn_head_kernel(x_ref, wq_ref, wk_ref, wv_ref, cnt_ref, ctx_ref):
    x = x_ref[...]
    q = jnp.dot(x, wq_ref[0], preferred_element_type=jnp.float32)
    q = q * (1.0 / math.sqrt(DK))
    k = jnp.dot(x, wk_ref[0], preferred_element_type=jnp.float32)
    v = jnp.dot(x, wv_ref[0], preferred_element_type=jnp.float32)

    KB = 512
    runmax = jnp.full((1, L), NEG, jnp.float32)
    runsum = jnp.zeros((1, L), jnp.float32)
    for b in range(L // KB):
        kb = k[b * KB:(b + 1) * KB, :]
        s = jax.lax.dot_general(kb, q, (((1,), (1,)), ((), ())),
                                preferred_element_type=jnp.float32)
        cnt = cnt_ref[b * KB:(b + 1) * KB, :]
        runmax = jnp.maximum(
            runmax, jnp.max(jnp.where(cnt > 0, s, NEG), axis=0, keepdims=True))
        runsum = runsum + jnp.sum(s * cnt, axis=0, keepdims=True)
    m_meas = runmax - runsum * (1.0 / L)

    iota = jax.lax.broadcasted_iota(jnp.int32, (1, L), 1)
    mv = m_meas
    rows = []
    for _ in range(U):
        mx = jnp.max(mv)
        amin = jnp.min(jnp.where(mv == mx, iota, L))
        row = iota == amin
        rows.append(row.astype(jnp.float32))
        mv = jnp.where(row, NEG, mv)
    onehot = jnp.concatenate(rows, axis=0)

    q_sel = jnp.dot(onehot, q, preferred_element_type=jnp.float32)
    scores = jax.lax.dot_general(q_sel, k, (((1,), (1,)), ((), ())),
                                 preferred_element_type=jnp.float32)
    smax = jnp.max(scores, axis=1, keepdims=True)
    e = jnp.exp(scores - smax)
    attn = e / jnp.sum(e, axis=1, keepdims=True)
    upd = jnp.dot(attn, v, preferred_element_type=jnp.float32)

    meanv = jnp.mean(v, axis=0, keepdims=True)
    delta = upd - meanv
    ctx_ref[0] = jnp.broadcast_to(meanv, (L, DK)) + jax.lax.dot_general(
        onehot, delta, (((0,), (0,)), ((), ())),
        preferred_element_type=jnp.float32)


def _out_kernel(ctx_ref, res_ref, wfc_ref, bfc_ref, g_ref, b_ref, o_ref):
    t = jnp.dot(ctx_ref[...], wfc_ref[...], preferred_element_type=jnp.float32)
    t = t + bfc_ref[...] + res_ref[...]
    mu = jnp.mean(t, axis=1, keepdims=True)
    d = t - mu
    var = jnp.mean(d * d, axis=1, keepdims=True)
    o_ref[...] = d * jax.lax.rsqrt(var + EPS) * g_ref[...] + b_ref[...]


def kernel(hidden_states, Wq, Wk, Wv, Wfc, bfc, gamma, beta):
    x = hidden_states.reshape(L, DM)
    cnt_t = jnp.asarray(_CNT_T)
    wq3 = Wq.reshape(DM, H, DK).transpose(1, 0, 2)
    wk3 = Wk.reshape(DM, H, DK).transpose(1, 0, 2)
    wv3 = Wv.reshape(DM, H, DK).transpose(1, 0, 2)

    ctx3 = pl.pallas_call(
        _attn_head_kernel,
        grid=(H,),
        in_specs=[
            pl.BlockSpec((L, DM), lambda h: (0, 0)),
            pl.BlockSpec((1, DM, DK), lambda h: (h, 0, 0)),
            pl.BlockSpec((1, DM, DK), lambda h: (h, 0, 0)),
            pl.BlockSpec((1, DM, DK), lambda h: (h, 0, 0)),
            pl.BlockSpec((L, L), lambda h: (0, 0)),
        ],
        out_specs=pl.BlockSpec((1, L, DK), lambda h: (h, 0, 0)),
        out_shape=jax.ShapeDtypeStruct((H, L, DK), jnp.float32),
    )(x, wq3, wk3, wv3, cnt_t)
    ctx = ctx3.transpose(1, 0, 2).reshape(L, H * DK)

    BL = 256
    out = pl.pallas_call(
        _out_kernel,
        grid=(L // BL,),
        in_specs=[
            pl.BlockSpec((BL, DM), lambda i: (i, 0)),
            pl.BlockSpec((BL, DM), lambda i: (i, 0)),
            pl.BlockSpec((DM, DM), lambda i: (0, 0)),
            pl.BlockSpec((1, DM), lambda i: (0, 0)),
            pl.BlockSpec((1, DM), lambda i: (0, 0)),
            pl.BlockSpec((1, DM), lambda i: (0, 0)),
        ],
        out_specs=pl.BlockSpec((BL, DM), lambda i: (i, 0)),
        out_shape=jax.ShapeDtypeStruct((L, DM), jnp.float32),
    )(ctx, x, Wfc, bfc.reshape(1, DM), gamma.reshape(1, DM),
      beta.reshape(1, DM))

    return out.reshape(1, L, DM)


# speedup vs baseline: 1.7495x; 1.7495x over previous
import math

import numpy as np
import jax
import jax.numpy as jnp
from jax.experimental import pallas as pl
from jax.experimental.pallas import tpu as pltpu

L = 2048
DM = 768
H = 12
DK = 64
U = min(5 * int(np.ceil(np.log(L))), L)
EPS = 1e-6
NEG = float(np.float32(-3.0e38))


def _rotl32(x, d):
    return ((x << np.uint32(d)) | (x >> np.uint32(32 - d))).astype(np.uint32)


def _threefry2x32(k0, k1, x0, x1):
    rot = [(13, 15, 26, 6), (17, 29, 16, 24)]
    ks = [np.uint32(k0), np.uint32(k1),
          np.uint32(np.uint32(k0) ^ np.uint32(k1) ^ np.uint32(0x1BD11BDA))]
    x0 = (x0 + ks[0]).astype(np.uint32)
    x1 = (x1 + ks[1]).astype(np.uint32)
    for i in range(5):
        for r in rot[i % 2]:
            x0 = (x0 + x1).astype(np.uint32)
            x1 = _rotl32(x1, r)
            x1 = (x1 ^ x0).astype(np.uint32)
        x0 = (x0 + ks[(i + 1) % 3]).astype(np.uint32)
        x1 = (x1 + ks[(i + 2) % 3] + np.uint32(i + 1)).astype(np.uint32)
    return x0, x1


def _sample_indices() -> np.ndarray:
    b1, b2 = _threefry2x32(0, 42, np.zeros(2, np.uint32),
                           np.arange(2, dtype=np.uint32))
    n = L * U
    h1, h2 = _threefry2x32(b1[1], b2[1], np.zeros(n, np.uint32),
                           np.arange(n, dtype=np.uint32))
    bits = (h1 ^ h2).astype(np.uint32)
    return (bits % np.uint32(L)).astype(np.int32).reshape(L, U)


def _sample_counts_T() -> np.ndarray:
    idx = _sample_indices()
    cnt = np.zeros((L, L), np.float32)
    np.add.at(cnt, (np.arange(L)[:, None], idx), 1.0)
    return np.ascontiguousarray(cnt.T)


_CNT_T = _sample_counts_T()


def _proj_score_kernel(x_ref, wq_ref, wk_ref, wv_ref, cnt_ref,
                       q_out, k_out, v_out, m_out):
    x = x_ref[...]
    q = jnp.dot(x, wq_ref[0], preferred_element_type=jnp.float32)
    q = q * (1.0 / math.sqrt(DK))
    k = jnp.dot(x, wk_ref[0], preferred_element_type=jnp.float32)
    v = jnp.dot(x, wv_ref[0], preferred_element_type=jnp.float32)
    q_out[0] = q
    k_out[0] = k
    v_out[0] = v

    KB = 512
    runmax = jnp.full((1, L), NEG, jnp.float32)
    runsum = jnp.zeros((1, L), jnp.float32)
    for b in range(L // KB):
        kb = k[b * KB:(b + 1) * KB, :]
        s = jax.lax.dot_general(kb, q, (((1,), (1,)), ((), ())),
                                preferred_element_type=jnp.float32)
        cnt = cnt_ref[b * KB:(b + 1) * KB, :]
        runmax = jnp.maximum(
            runmax, jnp.max(jnp.where(cnt > 0, s, NEG), axis=0, keepdims=True))
        runsum = runsum + jnp.sum(s * cnt, axis=0, keepdims=True)
    m_out[0] = runmax - runsum * (1.0 / L)


def _topk_kernel(m_ref, idx_ref):
    mv = m_ref[...]
    iota = jax.lax.broadcasted_iota(jnp.int32, (H, L), 1)
    for r in range(U):
        mx = jnp.max(mv, axis=1, keepdims=True)
        eq = mv == mx
        amin = jnp.min(jnp.where(eq, iota, L), axis=1, keepdims=True)
        idx_ref[:, r:r + 1] = amin
        mv = jnp.where(iota == amin, NEG, mv)


def _attn_apply_kernel(idx_sref, q_ref, k_ref, v_ref, ctx_ref):
    h = pl.program_id(0)
    q = q_ref[0]
    k = k_ref[0]
    v = v_ref[0]
    iota = jax.lax.broadcasted_iota(jnp.int32, (1, L), 1)
    rows = [(iota == idx_sref[h * U + r]).astype(jnp.float32)
            for r in range(U)]
    onehot = jnp.concatenate(rows, axis=0)

    q_sel = jnp.dot(onehot, q, preferred_element_type=jnp.float32)
    scores = jax.lax.dot_general(q_sel, k, (((1,), (1,)), ((), ())),
                                 preferred_element_type=jnp.float32)
    smax = jnp.max(scores, axis=1, keepdims=True)
    e = jnp.exp(scores - smax)
    attn = e / jnp.sum(e, axis=1, keepdims=True)
    upd = jnp.dot(attn, v, preferred_element_type=jnp.float32)

    meanv = jnp.mean(v, axis=0, keepdims=True)
    delta = upd - meanv
    ctx_ref[0] = jnp.broadcast_to(meanv, (L, DK)) + jax.lax.dot_general(
        onehot, delta, (((0,), (0,)), ((), ())),
        preferred_element_type=jnp.float32)


def _out_kernel(ctx_ref, res_ref, wfc_ref, bfc_ref, g_ref, b_ref, o_ref):
    t = jnp.dot(ctx_ref[...], wfc_ref[...], preferred_element_type=jnp.float32)
    t = t + bfc_ref[...] + res_ref[...]
    mu = jnp.mean(t, axis=1, keepdims=True)
    d = t - mu
    var = jnp.mean(d * d, axis=1, keepdims=True)
    o_ref[...] = d * jax.lax.rsqrt(var + EPS) * g_ref[...] + b_ref[...]


def kernel(hidden_states, Wq, Wk, Wv, Wfc, bfc, gamma, beta):
    x = hidden_states.reshape(L, DM)
    cnt_t = jnp.asarray(_CNT_T)
    wq3 = Wq.reshape(DM, H, DK).transpose(1, 0, 2)
    wk3 = Wk.reshape(DM, H, DK).transpose(1, 0, 2)
    wv3 = Wv.reshape(DM, H, DK).transpose(1, 0, 2)

    q3, k3, v3, m3 = pl.pallas_call(
        _proj_score_kernel,
        grid=(H,),
        in_specs=[
            pl.BlockSpec((L, DM), lambda h: (0, 0)),
            pl.BlockSpec((1, DM, DK), lambda h: (h, 0, 0)),
            pl.BlockSpec((1, DM, DK), lambda h: (h, 0, 0)),
            pl.BlockSpec((1, DM, DK), lambda h: (h, 0, 0)),
            pl.BlockSpec((L, L), lambda h: (0, 0)),
        ],
        out_specs=[
            pl.BlockSpec((1, L, DK), lambda h: (h, 0, 0)),
            pl.BlockSpec((1, L, DK), lambda h: (h, 0, 0)),
            pl.BlockSpec((1, L, DK), lambda h: (h, 0, 0)),
            pl.BlockSpec((1, 1, L), lambda h: (h, 0, 0)),
        ],
        out_shape=[
            jax.ShapeDtypeStruct((H, L, DK), jnp.float32),
            jax.ShapeDtypeStruct((H, L, DK), jnp.float32),
            jax.ShapeDtypeStruct((H, L, DK), jnp.float32),
            jax.ShapeDtypeStruct((H, 1, L), jnp.float32),
        ],
    )(x, wq3, wk3, wv3, cnt_t)

    idx = pl.pallas_call(
        _topk_kernel,
        out_shape=jax.ShapeDtypeStruct((H, U), jnp.int32),
    )(m3.reshape(H, L))

    ctx3 = pl.pallas_call(
        _attn_apply_kernel,
        grid_spec=pltpu.PrefetchScalarGridSpec(
            num_scalar_prefetch=1,
            grid=(H,),
            in_specs=[
                pl.BlockSpec((1, L, DK), lambda h, idx_sref: (h, 0, 0)),
                pl.BlockSpec((1, L, DK), lambda h, idx_sref: (h, 0, 0)),
                pl.BlockSpec((1, L, DK), lambda h, idx_sref: (h, 0, 0)),
            ],
            out_specs=pl.BlockSpec((1, L, DK), lambda h, idx_sref: (h, 0, 0)),
        ),
        out_shape=jax.ShapeDtypeStruct((H, L, DK), jnp.float32),
    )(idx.reshape(H * U), q3, k3, v3)

    ctx = ctx3.transpose(1, 0, 2).reshape(L, H * DK)

    BL = 256
    out = pl.pallas_call(
        _out_kernel,
        grid=(L // BL,),
        in_specs=[
            pl.BlockSpec((BL, DM), lambda i: (i, 0)),
            pl.BlockSpec((BL, DM), lambda i: (i, 0)),
            pl.BlockSpec((DM, DM), lambda i: (0, 0)),
            pl.BlockSpec((1, DM), lambda i: (0, 0)),
            pl.BlockSpec((1, DM), lambda i: (0, 0)),
            pl.BlockSpec((1, DM), lambda i: (0, 0)),
        ],
        out_specs=pl.BlockSpec((BL, DM), lambda i: (i, 0)),
        out_shape=jax.ShapeDtypeStruct((L, DM), jnp.float32),
    )(ctx, x, Wfc, bfc.reshape(1, DM), gamma.reshape(1, DM),
      beta.reshape(1, DM))

    return out.reshape(1, L, DM)
